# baseline (speedup 1.0000x reference)
import functools

import jax
import jax.numpy as jnp
from jax import lax
from jax.experimental import pallas as pl
from jax.experimental.pallas import tpu as pltpu
from jax.experimental.pallas import tpu_sc as plsc

_TOPK = 64
_POS_INF_BITS = 0x7F800000


def _enc_kernel(x_ref, w_ref, b_ref, h_ref):
    acc = jax.lax.dot_general(
        x_ref[...].astype(jnp.bfloat16),
        w_ref[...].astype(jnp.bfloat16),
        (((1,), (1,)), ((), ())),
        preferred_element_type=jnp.float32,
    )
    h_ref[...] = acc + b_ref[...]


def _thresh_kernel(h_ref, z_ref, *, k):
    hv = h_ref[...]
    bm = hv.shape[0]
    lo = jnp.zeros((bm, 1), jnp.int32)
    hi = jnp.full((bm, 1), _POS_INF_BITS, jnp.int32)

    def body(_, carry):
        lo, hi = carry
        mid = (lo + hi) >> 1
        t = jax.lax.bitcast_convert_type(mid, jnp.float32)
        cnt = jnp.sum((hv >= t).astype(jnp.float32), axis=1, keepdims=True)
        ge = cnt >= k
        return jnp.where(ge, mid, lo), jnp.where(ge, hi, mid)

    lo, hi = jax.lax.fori_loop(0, 31, body, (lo, hi))
    t = jax.lax.bitcast_convert_type(lo, jnp.float32)
    mask = (hv >= t) & (hv > 0.0)
    z_ref[...] = jnp.where(mask, hv, 0.0)


def _sc_thresh_body(h_hbm, z_hbm, hrow, zrow, hist, cand):
    i32 = jnp.int32
    nrows, d_lat = h_hbm.shape
    info = plsc.get_sparse_core_info()
    nw = info.num_cores * info.num_subcores
    wid = lax.axis_index("s") * info.num_cores + lax.axis_index("c")
    rows_per_w = nrows // nw
    nchunk = d_lat // 16

    lanes = lax.iota(i32, 16)
    lane_base = lanes * 256
    ones16 = jnp.ones((16,), i32)
    zeros16 = jnp.zeros((16,), i32)

    def _clr(k, _):
        hist[pl.ds(k * 16, 16)] = zeros16
        return 0

    lax.fori_loop(0, 256, _clr, 0)

    def row_body(i, _):
        row = wid * rows_per_w + i
        pltpu.sync_copy(h_hbm.at[row], hrow)

        def p1(j, _c):
            hv = hrow[pl.ds(j * 16, 16)]
            u = lax.bitcast_convert_type(hv, i32)
            upos = jnp.maximum(u, 0)
            e = lax.shift_right_logical(upos, 23)
            plsc.addupdate_scatter(hist, [lane_base + e], ones16)
            return 0

        lax.fori_loop(0, nchunk, p1, 0)

        def p2(k, carry):
            carry_cnt, bstar, cnt_above = carry
            c = 15 - k
            acc = zeros16
            for l in range(16):
                sl = pl.ds(l * 256 + c * 16, 16)
                acc = acc + hist[sl]
                hist[sl] = zeros16
            rev = lax.rev(acc, (0,))
            cum = plsc.cumsum(rev) + carry_cnt
            prev = cum - rev
            first = (cum >= _TOPK) & (prev < _TOPK)
            bucket_ids = c * 16 + 15 - lanes
            bstar = bstar + jnp.sum(jnp.where(first, bucket_ids, 0))
            cnt_above = cnt_above + jnp.sum(jnp.where(first, prev, 0))
            carry_cnt = carry_cnt + jnp.sum(acc)
            return carry_cnt, bstar, cnt_above

        _, bstar, cnt_above = lax.fori_loop(
            0, 16, p2, (jnp.int32(0), jnp.int32(0), jnp.int32(0)))
        r_needed = _TOPK - cnt_above

        def p3(j, off):
            hv = hrow[pl.ds(j * 16, 16)]
            u = lax.bitcast_convert_type(hv, i32)
            upos = jnp.maximum(u, 0)
            e = lax.shift_right_logical(upos, 23)
            m = e == bstar
            mi = m.astype(i32)
            pos = plsc.cumsum(mi) - mi
            plsc.store_scatter(cand, [off + pos], upos, mask=m)
            return off + plsc.all_reduce_population_count(m)

        off = lax.fori_loop(0, nchunk, p3, zeros16)
        plsc.store_scatter(cand, [off + lanes], zeros16)
        nc = jnp.max(off)
        nch = (nc + 15) // 16

        base = bstar << 23

        def p4(_it, carry):
            lo_d, hi_d = carry
            mid_d = (lo_d + hi_d) >> 1
            tmid = base + mid_d

            def inner(q, acc):
                cv = cand[pl.ds(q * 16, 16)]
                return acc + (cv >= tmid).astype(i32)

            cnt = jnp.sum(lax.fori_loop(0, nch, inner, zeros16))
            ge = cnt >= r_needed
            return (jnp.where(ge, mid_d, lo_d), jnp.where(ge, hi_d, mid_d))

        lo_d, _hi = lax.fori_loop(0, 23, p4, (jnp.int32(0), jnp.int32(1 << 23)))
        tbits = base + lo_d
        tvec = lax.bitcast_convert_type(jnp.full((16,), tbits, i32), jnp.float32)

        def p5(j, _c):
            hv = hrow[pl.ds(j * 16, 16)]
            m = (hv >= tvec) & (hv > 0.0)
            zrow[pl.ds(j * 16, 16)] = jnp.where(m, hv, jnp.float32(0.0))
            return 0

        lax.fori_loop(0, nchunk, p5, 0)
        pltpu.sync_copy(zrow, z_hbm.at[row])
        return 0

    lax.fori_loop(0, rows_per_w, row_body, 0)


def _sc_thresh(h):
    b, d_lat = h.shape
    mesh = plsc.VectorSubcoreMesh(core_axis_name="c", subcore_axis_name="s")
    return pl.kernel(
        _sc_thresh_body,
        out_type=jax.ShapeDtypeStruct((b, d_lat), jnp.float32),
        mesh=mesh,
        compiler_params=pltpu.CompilerParams(needs_layout_passes=False),
        scratch_types=[
            pltpu.VMEM((d_lat,), jnp.float32),
            pltpu.VMEM((d_lat,), jnp.float32),
            pltpu.VMEM((4096,), jnp.int32),
            pltpu.VMEM((d_lat + 16,), jnp.int32),
        ],
    )(h)


def _dec_kernel(z_ref, w_ref, o_ref):
    w = w_ref[...]
    sq = jnp.sum(w * w, axis=0, keepdims=True)
    inv_s = 1.0 / jnp.maximum(jnp.sqrt(sq), 1e-8)
    zk = (z_ref[...] * inv_s).astype(jnp.bfloat16)
    part = jax.lax.dot_general(
        zk, w.astype(jnp.bfloat16), (((1,), (1,)), ((), ())),
        preferred_element_type=jnp.float32,
    )

    @pl.when(pl.program_id(1) == 0)
    def _():
        o_ref[...] = part

    @pl.when(pl.program_id(1) != 0)
    def _():
        o_ref[...] += part


@jax.jit
def kernel(x, W_enc, b_enc, W_dec):
    b, d_in = x.shape
    d_lat = W_enc.shape[0]
    f32 = jnp.float32

    bm1 = min(1024, b)
    bn1 = min(1024, d_lat)
    h = pl.pallas_call(
        _enc_kernel,
        grid=(b // bm1, d_lat // bn1),
        in_specs=[
            pl.BlockSpec((bm1, d_in), lambda i, j: (i, 0)),
            pl.BlockSpec((bn1, d_in), lambda i, j: (j, 0)),
            pl.BlockSpec((1, bn1), lambda i, j: (0, j)),
        ],
        out_specs=pl.BlockSpec((bm1, bn1), lambda i, j: (i, j)),
        out_shape=jax.ShapeDtypeStruct((b, d_lat), f32),
        compiler_params=pltpu.CompilerParams(
            dimension_semantics=("parallel", "parallel"),
        ),
    )(x, W_enc, b_enc.reshape(1, d_lat))

    z = _sc_thresh(h)

    bm4 = min(1024, b)
    bk4 = min(512, d_lat)
    x_hat = pl.pallas_call(
        _dec_kernel,
        grid=(b // bm4, d_lat // bk4),
        in_specs=[
            pl.BlockSpec((bm4, bk4), lambda i, k: (i, k)),
            pl.BlockSpec((d_in, bk4), lambda i, k: (0, k)),
        ],
        out_specs=pl.BlockSpec((bm4, d_in), lambda i, k: (i, 0)),
        out_shape=jax.ShapeDtypeStruct((b, d_in), f32),
        compiler_params=pltpu.CompilerParams(
            dimension_semantics=("parallel", "arbitrary"),
        ),
    )(z, W_dec)

    return x_hat, z


# speedup vs baseline: 2.5526x; 2.5526x over previous
import functools

import jax
import jax.numpy as jnp
from jax import lax
from jax.experimental import pallas as pl
from jax.experimental.pallas import tpu as pltpu
from jax.experimental.pallas import tpu_sc as plsc

_TOPK = 64
_POS_INF_BITS = 0x7F800000


def _enc_kernel(x_ref, w_ref, b_ref, h_ref):
    acc = jax.lax.dot_general(
        x_ref[...].astype(jnp.bfloat16),
        w_ref[...].astype(jnp.bfloat16),
        (((1,), (1,)), ((), ())),
        preferred_element_type=jnp.float32,
    )
    h_ref[...] = acc + b_ref[...]


def _thresh_kernel(h_ref, z_ref, *, k):
    hv = h_ref[...]
    bm = hv.shape[0]
    lo = jnp.zeros((bm, 1), jnp.int32)
    hi = jnp.full((bm, 1), _POS_INF_BITS, jnp.int32)

    ones8 = jnp.ones((hv.shape[1], 8), jnp.float32)

    def body(_, carry):
        lo, hi = carry
        mid = (lo + hi) >> 1
        t = jax.lax.bitcast_convert_type(mid, jnp.float32)
        mf = jnp.where(hv >= t, 1.0, 0.0)
        cnt = jax.lax.dot_general(
            mf, ones8, (((1,), (0,)), ((), ())),
            preferred_element_type=jnp.float32)[:, :1]
        ge = cnt >= k
        return jnp.where(ge, mid, lo), jnp.where(ge, hi, mid)

    lo, hi = jax.lax.fori_loop(0, 31, body, (lo, hi))
    t = jax.lax.bitcast_convert_type(lo, jnp.float32)
    mask = (hv >= t) & (hv > 0.0)
    z_ref[...] = jnp.where(mask, hv, 0.0)


def _sc_thresh_body(h_hbm, z_hbm, hrow, zrow, hist, cand):
    i32 = jnp.int32
    nrows, d_lat = h_hbm.shape
    info = plsc.get_sparse_core_info()
    nw = info.num_cores * info.num_subcores
    wid = lax.axis_index("s") * info.num_cores + lax.axis_index("c")
    rows_per_w = nrows // nw
    nchunk = d_lat // 16

    lanes = lax.iota(i32, 16)
    lane_base = lanes * 256
    ones16 = jnp.ones((16,), i32)
    zeros16 = jnp.zeros((16,), i32)

    def _clr(k, _):
        hist[pl.ds(k * 16, 16)] = zeros16
        return 0

    lax.fori_loop(0, 256, _clr, 0)

    def row_body(i, _):
        row = wid * rows_per_w + i
        pltpu.sync_copy(h_hbm.at[row], hrow)

        def p1(j, _c):
            hv = hrow[pl.ds(j * 16, 16)]
            u = lax.bitcast_convert_type(hv, i32)
            upos = jnp.maximum(u, 0)
            e = lax.shift_right_logical(upos, 23)
            plsc.addupdate_scatter(hist, [lane_base + e], ones16)
            return 0

        lax.fori_loop(0, nchunk, p1, 0)

        def p2(k, carry):
            carry_cnt, bstar, cnt_above = carry
            c = 15 - k
            acc = zeros16
            for l in range(16):
                sl = pl.ds(l * 256 + c * 16, 16)
                acc = acc + hist[sl]
                hist[sl] = zeros16
            rev = lax.rev(acc, (0,))
            cum = plsc.cumsum(rev) + carry_cnt
            prev = cum - rev
            first = (cum >= _TOPK) & (prev < _TOPK)
            bucket_ids = c * 16 + 15 - lanes
            bstar = bstar + jnp.sum(jnp.where(first, bucket_ids, 0))
            cnt_above = cnt_above + jnp.sum(jnp.where(first, prev, 0))
            carry_cnt = carry_cnt + jnp.sum(acc)
            return carry_cnt, bstar, cnt_above

        _, bstar, cnt_above = lax.fori_loop(
            0, 16, p2, (jnp.int32(0), jnp.int32(0), jnp.int32(0)))
        r_needed = _TOPK - cnt_above

        def p3(j, off):
            hv = hrow[pl.ds(j * 16, 16)]
            u = lax.bitcast_convert_type(hv, i32)
            upos = jnp.maximum(u, 0)
            e = lax.shift_right_logical(upos, 23)
            m = e == bstar
            mi = m.astype(i32)
            pos = plsc.cumsum(mi) - mi
            plsc.store_scatter(cand, [off + pos], upos, mask=m)
            return off + plsc.all_reduce_population_count(m)

        off = lax.fori_loop(0, nchunk, p3, zeros16)
        plsc.store_scatter(cand, [off + lanes], zeros16)
        nc = jnp.max(off)
        nch = (nc + 15) // 16

        base = bstar << 23

        def p4(_it, carry):
            lo_d, hi_d = carry
            mid_d = (lo_d + hi_d) >> 1
            tmid = base + mid_d

            def inner(q, acc):
                cv = cand[pl.ds(q * 16, 16)]
                return acc + (cv >= tmid).astype(i32)

            cnt = jnp.sum(lax.fori_loop(0, nch, inner, zeros16))
            ge = cnt >= r_needed
            return (jnp.where(ge, mid_d, lo_d), jnp.where(ge, hi_d, mid_d))

        lo_d, _hi = lax.fori_loop(0, 23, p4, (jnp.int32(0), jnp.int32(1 << 23)))
        tbits = base + lo_d
        tvec = lax.bitcast_convert_type(jnp.full((16,), tbits, i32), jnp.float32)

        def p5(j, _c):
            hv = hrow[pl.ds(j * 16, 16)]
            m = (hv >= tvec) & (hv > 0.0)
            zrow[pl.ds(j * 16, 16)] = jnp.where(m, hv, jnp.float32(0.0))
            return 0

        lax.fori_loop(0, nchunk, p5, 0)
        pltpu.sync_copy(zrow, z_hbm.at[row])
        return 0

    lax.fori_loop(0, rows_per_w, row_body, 0)


def _sc_thresh(h):
    b, d_lat = h.shape
    mesh = plsc.VectorSubcoreMesh(core_axis_name="c", subcore_axis_name="s")
    return pl.kernel(
        _sc_thresh_body,
        out_type=jax.ShapeDtypeStruct((b, d_lat), jnp.float32),
        mesh=mesh,
        compiler_params=pltpu.CompilerParams(needs_layout_passes=False),
        scratch_types=[
            pltpu.VMEM((d_lat,), jnp.float32),
            pltpu.VMEM((d_lat,), jnp.float32),
            pltpu.VMEM((4096,), jnp.int32),
            pltpu.VMEM((d_lat + 16,), jnp.int32),
        ],
    )(h)


def _dec_kernel(z_ref, w_ref, o_ref):
    w = w_ref[...]
    sq = jnp.sum(w * w, axis=0, keepdims=True)
    inv_s = 1.0 / jnp.maximum(jnp.sqrt(sq), 1e-8)
    zk = (z_ref[...] * inv_s).astype(jnp.bfloat16)
    part = jax.lax.dot_general(
        zk, w.astype(jnp.bfloat16), (((1,), (1,)), ((), ())),
        preferred_element_type=jnp.float32,
    )

    @pl.when(pl.program_id(1) == 0)
    def _():
        o_ref[...] = part

    @pl.when(pl.program_id(1) != 0)
    def _():
        o_ref[...] += part


@jax.jit
def kernel(x, W_enc, b_enc, W_dec):
    b, d_in = x.shape
    d_lat = W_enc.shape[0]
    f32 = jnp.float32

    bm1 = min(2048, b)
    bn1 = min(512, d_lat)
    h = pl.pallas_call(
        _enc_kernel,
        grid=(b // bm1, d_lat // bn1),
        in_specs=[
            pl.BlockSpec((bm1, d_in), lambda i, j: (i, 0)),
            pl.BlockSpec((bn1, d_in), lambda i, j: (j, 0)),
            pl.BlockSpec((1, bn1), lambda i, j: (0, j)),
        ],
        out_specs=pl.BlockSpec((bm1, bn1), lambda i, j: (i, j)),
        out_shape=jax.ShapeDtypeStruct((b, d_lat), f32),
        compiler_params=pltpu.CompilerParams(
            dimension_semantics=("parallel", "parallel"),
        ),
    )(x, W_enc, b_enc.reshape(1, d_lat))

    bm2 = min(128, b)
    z = pl.pallas_call(
        functools.partial(_thresh_kernel, k=_TOPK),
        grid=(b // bm2,),
        in_specs=[pl.BlockSpec((bm2, d_lat), lambda i: (i, 0))],
        out_specs=pl.BlockSpec((bm2, d_lat), lambda i: (i, 0)),
        out_shape=jax.ShapeDtypeStruct((b, d_lat), f32),
        compiler_params=pltpu.CompilerParams(
            dimension_semantics=("parallel",),
        ),
    )(h)

    bm4 = min(1024, b)
    bk4 = min(512, d_lat)
    x_hat = pl.pallas_call(
        _dec_kernel,
        grid=(b // bm4, d_lat // bk4),
        in_specs=[
            pl.BlockSpec((bm4, bk4), lambda i, k: (i, k)),
            pl.BlockSpec((d_in, bk4), lambda i, k: (0, k)),
        ],
        out_specs=pl.BlockSpec((bm4, d_in), lambda i, k: (i, 0)),
        out_shape=jax.ShapeDtypeStruct((b, d_in), f32),
        compiler_params=pltpu.CompilerParams(
            dimension_semantics=("parallel", "arbitrary"),
        ),
    )(z, W_dec)

    return x_hat, z


# speedup vs baseline: 3.0617x; 1.1994x over previous
import functools

import jax
import jax.numpy as jnp
from jax import lax
from jax.experimental import pallas as pl
from jax.experimental.pallas import tpu as pltpu
from jax.experimental.pallas import tpu_sc as plsc

_TOPK = 64
_POS_INF_BITS = 0x7F800000


def _enc_kernel(x_ref, w_ref, b_ref, h_ref):
    acc = jax.lax.dot_general(
        x_ref[...].astype(jnp.bfloat16),
        w_ref[...].astype(jnp.bfloat16),
        (((1,), (1,)), ((), ())),
        preferred_element_type=jnp.float32,
    )
    h_ref[...] = acc + b_ref[...]


def _thresh_kernel(h_ref, z_ref, *, k):
    hv = h_ref[...]
    bm = hv.shape[0]
    lo = jnp.zeros((bm, 1), jnp.int32)
    hi = jnp.full((bm, 1), _POS_INF_BITS, jnp.int32)

    def body(_, carry):
        lo, hi = carry
        mid = (lo + hi) >> 1
        t = jax.lax.bitcast_convert_type(mid, jnp.float32)
        cnt = jnp.sum((hv >= t).astype(jnp.float32), axis=1, keepdims=True)
        ge = cnt >= k
        return jnp.where(ge, mid, lo), jnp.where(ge, hi, mid)

    lo, hi = jax.lax.fori_loop(0, 31, body, (lo, hi))
    t = jax.lax.bitcast_convert_type(lo, jnp.float32)
    mask = (hv >= t) & (hv > 0.0)
    z_ref[...] = jnp.where(mask, hv, 0.0)


def _sc_thresh_body(h_hbm, z_hbm, hrow, zrow, hist, cand):
    i32 = jnp.int32
    nrows, d_lat = h_hbm.shape
    info = plsc.get_sparse_core_info()
    nw = info.num_cores * info.num_subcores
    wid = lax.axis_index("s") * info.num_cores + lax.axis_index("c")
    rows_per_w = nrows // nw
    nchunk = d_lat // 16

    lanes = lax.iota(i32, 16)
    lane_base = lanes * 256
    ones16 = jnp.ones((16,), i32)
    zeros16 = jnp.zeros((16,), i32)

    def _clr(k, _):
        hist[pl.ds(k * 16, 16)] = zeros16
        return 0

    lax.fori_loop(0, 256, _clr, 0)

    def row_body(i, _):
        row = wid * rows_per_w + i
        pltpu.sync_copy(h_hbm.at[row], hrow)

        def p1(j, _c):
            hv = hrow[pl.ds(j * 16, 16)]
            u = lax.bitcast_convert_type(hv, i32)
            upos = jnp.maximum(u, 0)
            e = lax.shift_right_logical(upos, 23)
            plsc.addupdate_scatter(hist, [lane_base + e], ones16)
            return 0

        lax.fori_loop(0, nchunk, p1, 0)

        def p2(k, carry):
            carry_cnt, bstar, cnt_above = carry
            c = 15 - k
            acc = zeros16
            for l in range(16):
                sl = pl.ds(l * 256 + c * 16, 16)
                acc = acc + hist[sl]
                hist[sl] = zeros16
            rev = lax.rev(acc, (0,))
            cum = plsc.cumsum(rev) + carry_cnt
            prev = cum - rev
            first = (cum >= _TOPK) & (prev < _TOPK)
            bucket_ids = c * 16 + 15 - lanes
            bstar = bstar + jnp.sum(jnp.where(first, bucket_ids, 0))
            cnt_above = cnt_above + jnp.sum(jnp.where(first, prev, 0))
            carry_cnt = carry_cnt + jnp.sum(acc)
            return carry_cnt, bstar, cnt_above

        _, bstar, cnt_above = lax.fori_loop(
            0, 16, p2, (jnp.int32(0), jnp.int32(0), jnp.int32(0)))
        r_needed = _TOPK - cnt_above

        def p3(j, off):
            hv = hrow[pl.ds(j * 16, 16)]
            u = lax.bitcast_convert_type(hv, i32)
            upos = jnp.maximum(u, 0)
            e = lax.shift_right_logical(upos, 23)
            m = e == bstar
            mi = m.astype(i32)
            pos = plsc.cumsum(mi) - mi
            plsc.store_scatter(cand, [off + pos], upos, mask=m)
            return off + plsc.all_reduce_population_count(m)

        off = lax.fori_loop(0, nchunk, p3, zeros16)
        plsc.store_scatter(cand, [off + lanes], zeros16)
        nc = jnp.max(off)
        nch = (nc + 15) // 16

        base = bstar << 23

        def p4(_it, carry):
            lo_d, hi_d = carry
            mid_d = (lo_d + hi_d) >> 1
            tmid = base + mid_d

            def inner(q, acc):
                cv = cand[pl.ds(q * 16, 16)]
                return acc + (cv >= tmid).astype(i32)

            cnt = jnp.sum(lax.fori_loop(0, nch, inner, zeros16))
            ge = cnt >= r_needed
            return (jnp.where(ge, mid_d, lo_d), jnp.where(ge, hi_d, mid_d))

        lo_d, _hi = lax.fori_loop(0, 23, p4, (jnp.int32(0), jnp.int32(1 << 23)))
        tbits = base + lo_d
        tvec = lax.bitcast_convert_type(jnp.full((16,), tbits, i32), jnp.float32)

        def p5(j, _c):
            hv = hrow[pl.ds(j * 16, 16)]
            m = (hv >= tvec) & (hv > 0.0)
            zrow[pl.ds(j * 16, 16)] = jnp.where(m, hv, jnp.float32(0.0))
            return 0

        lax.fori_loop(0, nchunk, p5, 0)
        pltpu.sync_copy(zrow, z_hbm.at[row])
        return 0

    lax.fori_loop(0, rows_per_w, row_body, 0)


def _sc_thresh(h):
    b, d_lat = h.shape
    mesh = plsc.VectorSubcoreMesh(core_axis_name="c", subcore_axis_name="s")
    return pl.kernel(
        _sc_thresh_body,
        out_type=jax.ShapeDtypeStruct((b, d_lat), jnp.float32),
        mesh=mesh,
        compiler_params=pltpu.CompilerParams(needs_layout_passes=False),
        scratch_types=[
            pltpu.VMEM((d_lat,), jnp.float32),
            pltpu.VMEM((d_lat,), jnp.float32),
            pltpu.VMEM((4096,), jnp.int32),
            pltpu.VMEM((d_lat + 16,), jnp.int32),
        ],
    )(h)


def _dec_kernel(z_ref, w_ref, o_ref):
    w = w_ref[...]
    sq = jnp.sum(w * w, axis=0, keepdims=True)
    inv_s = 1.0 / jnp.maximum(jnp.sqrt(sq), 1e-8)
    zk = (z_ref[...] * inv_s).astype(jnp.bfloat16)
    part = jax.lax.dot_general(
        zk, w.astype(jnp.bfloat16), (((1,), (1,)), ((), ())),
        preferred_element_type=jnp.float32,
    )

    @pl.when(pl.program_id(1) == 0)
    def _():
        o_ref[...] = part

    @pl.when(pl.program_id(1) != 0)
    def _():
        o_ref[...] += part


@jax.jit
def kernel(x, W_enc, b_enc, W_dec):
    b, d_in = x.shape
    d_lat = W_enc.shape[0]
    f32 = jnp.float32

    bm1 = min(2048, b)
    bn1 = min(512, d_lat)
    h = pl.pallas_call(
        _enc_kernel,
        grid=(b // bm1, d_lat // bn1),
        in_specs=[
            pl.BlockSpec((bm1, d_in), lambda i, j: (i, 0)),
            pl.BlockSpec((bn1, d_in), lambda i, j: (j, 0)),
            pl.BlockSpec((1, bn1), lambda i, j: (0, j)),
        ],
        out_specs=pl.BlockSpec((bm1, bn1), lambda i, j: (i, j)),
        out_shape=jax.ShapeDtypeStruct((b, d_lat), f32),
        compiler_params=pltpu.CompilerParams(
            dimension_semantics=("parallel", "parallel"),
        ),
    )(x, W_enc, b_enc.reshape(1, d_lat))

    bm2 = min(128, b)
    z = pl.pallas_call(
        functools.partial(_thresh_kernel, k=_TOPK),
        grid=(b // bm2,),
        in_specs=[pl.BlockSpec((bm2, d_lat), lambda i: (i, 0))],
        out_specs=pl.BlockSpec((bm2, d_lat), lambda i: (i, 0)),
        out_shape=jax.ShapeDtypeStruct((b, d_lat), f32),
        compiler_params=pltpu.CompilerParams(
            dimension_semantics=("parallel",),
        ),
    )(h)

    bm4 = min(1024, b)
    bk4 = min(512, d_lat)
    x_hat = pl.pallas_call(
        _dec_kernel,
        grid=(b // bm4, d_lat // bk4),
        in_specs=[
            pl.BlockSpec((bm4, bk4), lambda i, k: (i, k)),
            pl.BlockSpec((d_in, bk4), lambda i, k: (0, k)),
        ],
        out_specs=pl.BlockSpec((bm4, d_in), lambda i, k: (i, 0)),
        out_shape=jax.ShapeDtypeStruct((b, d_in), f32),
        compiler_params=pltpu.CompilerParams(
            dimension_semantics=("parallel", "arbitrary"),
        ),
    )(z, W_dec)

    return x_hat, z


# speedup vs baseline: 3.0637x; 1.0007x over previous
import functools

import jax
import jax.numpy as jnp
from jax import lax
from jax.experimental import pallas as pl
from jax.experimental.pallas import tpu as pltpu
from jax.experimental.pallas import tpu_sc as plsc

_TOPK = 64
_POS_INF_BITS = 0x7F800000


def _enc_kernel(x_ref, w_ref, b_ref, h_ref):
    acc = jax.lax.dot_general(
        x_ref[...].astype(jnp.bfloat16),
        w_ref[...].astype(jnp.bfloat16),
        (((1,), (1,)), ((), ())),
        preferred_element_type=jnp.float32,
    )
    h_ref[...] = acc + b_ref[...]


def _thresh_kernel(h_ref, z_ref, *, k):
    hv = h_ref[...]
    bm = hv.shape[0]
    lo = jnp.zeros((bm, 1), jnp.int32)
    hi = jnp.full((bm, 1), _POS_INF_BITS, jnp.int32)

    def body(_, carry):
        lo, hi = carry
        mid = (lo + hi) >> 1
        t = jax.lax.bitcast_convert_type(mid, jnp.float32)
        cnt = jnp.sum((hv >= t).astype(jnp.float32), axis=1, keepdims=True)
        ge = cnt >= k
        return jnp.where(ge, mid, lo), jnp.where(ge, hi, mid)

    lo, hi = jax.lax.fori_loop(0, 31, body, (lo, hi))
    t = jax.lax.bitcast_convert_type(lo, jnp.float32)
    mask = (hv >= t) & (hv > 0.0)
    z_ref[...] = jnp.where(mask, hv, 0.0)


def _sc_thresh_body(h_hbm, z_hbm, hrow, zrow, hist, cand):
    i32 = jnp.int32
    nrows, d_lat = h_hbm.shape
    info = plsc.get_sparse_core_info()
    nw = info.num_cores * info.num_subcores
    wid = lax.axis_index("s") * info.num_cores + lax.axis_index("c")
    rows_per_w = nrows // nw
    nchunk = d_lat // 16

    lanes = lax.iota(i32, 16)
    lane_base = lanes * 256
    ones16 = jnp.ones((16,), i32)
    zeros16 = jnp.zeros((16,), i32)

    def _clr(k, _):
        hist[pl.ds(k * 16, 16)] = zeros16
        return 0

    lax.fori_loop(0, 256, _clr, 0)

    def row_body(i, _):
        row = wid * rows_per_w + i
        pltpu.sync_copy(h_hbm.at[row], hrow)

        def p1(j, _c):
            hv = hrow[pl.ds(j * 16, 16)]
            u = lax.bitcast_convert_type(hv, i32)
            upos = jnp.maximum(u, 0)
            e = lax.shift_right_logical(upos, 23)
            plsc.addupdate_scatter(hist, [lane_base + e], ones16)
            return 0

        lax.fori_loop(0, nchunk, p1, 0)

        def p2(k, carry):
            carry_cnt, bstar, cnt_above = carry
            c = 15 - k
            acc = zeros16
            for l in range(16):
                sl = pl.ds(l * 256 + c * 16, 16)
                acc = acc + hist[sl]
                hist[sl] = zeros16
            rev = lax.rev(acc, (0,))
            cum = plsc.cumsum(rev) + carry_cnt
            prev = cum - rev
            first = (cum >= _TOPK) & (prev < _TOPK)
            bucket_ids = c * 16 + 15 - lanes
            bstar = bstar + jnp.sum(jnp.where(first, bucket_ids, 0))
            cnt_above = cnt_above + jnp.sum(jnp.where(first, prev, 0))
            carry_cnt = carry_cnt + jnp.sum(acc)
            return carry_cnt, bstar, cnt_above

        _, bstar, cnt_above = lax.fori_loop(
            0, 16, p2, (jnp.int32(0), jnp.int32(0), jnp.int32(0)))
        r_needed = _TOPK - cnt_above

        def p3(j, off):
            hv = hrow[pl.ds(j * 16, 16)]
            u = lax.bitcast_convert_type(hv, i32)
            upos = jnp.maximum(u, 0)
            e = lax.shift_right_logical(upos, 23)
            m = e == bstar
            mi = m.astype(i32)
            pos = plsc.cumsum(mi) - mi
            plsc.store_scatter(cand, [off + pos], upos, mask=m)
            return off + plsc.all_reduce_population_count(m)

        off = lax.fori_loop(0, nchunk, p3, zeros16)
        plsc.store_scatter(cand, [off + lanes], zeros16)
        nc = jnp.max(off)
        nch = (nc + 15) // 16

        base = bstar << 23

        def p4(_it, carry):
            lo_d, hi_d = carry
            mid_d = (lo_d + hi_d) >> 1
            tmid = base + mid_d

            def inner(q, acc):
                cv = cand[pl.ds(q * 16, 16)]
                return acc + (cv >= tmid).astype(i32)

            cnt = jnp.sum(lax.fori_loop(0, nch, inner, zeros16))
            ge = cnt >= r_needed
            return (jnp.where(ge, mid_d, lo_d), jnp.where(ge, hi_d, mid_d))

        lo_d, _hi = lax.fori_loop(0, 23, p4, (jnp.int32(0), jnp.int32(1 << 23)))
        tbits = base + lo_d
        tvec = lax.bitcast_convert_type(jnp.full((16,), tbits, i32), jnp.float32)

        def p5(j, _c):
            hv = hrow[pl.ds(j * 16, 16)]
            m = (hv >= tvec) & (hv > 0.0)
            zrow[pl.ds(j * 16, 16)] = jnp.where(m, hv, jnp.float32(0.0))
            return 0

        lax.fori_loop(0, nchunk, p5, 0)
        pltpu.sync_copy(zrow, z_hbm.at[row])
        return 0

    lax.fori_loop(0, rows_per_w, row_body, 0)


def _sc_thresh(h):
    b, d_lat = h.shape
    mesh = plsc.VectorSubcoreMesh(core_axis_name="c", subcore_axis_name="s")
    return pl.kernel(
        _sc_thresh_body,
        out_type=jax.ShapeDtypeStruct((b, d_lat), jnp.float32),
        mesh=mesh,
        compiler_params=pltpu.CompilerParams(needs_layout_passes=False),
        scratch_types=[
            pltpu.VMEM((d_lat,), jnp.float32),
            pltpu.VMEM((d_lat,), jnp.float32),
            pltpu.VMEM((4096,), jnp.int32),
            pltpu.VMEM((d_lat + 16,), jnp.int32),
        ],
    )(h)


def _dec_kernel(z_ref, w_ref, o_ref):
    w = w_ref[...]
    sq = jnp.sum(w * w, axis=0, keepdims=True)
    inv_s = 1.0 / jnp.maximum(jnp.sqrt(sq), 1e-8)
    zk = (z_ref[...] * inv_s).astype(jnp.bfloat16)
    part = jax.lax.dot_general(
        zk, w.astype(jnp.bfloat16), (((1,), (1,)), ((), ())),
        preferred_element_type=jnp.float32,
    )

    @pl.when(pl.program_id(1) == 0)
    def _():
        o_ref[...] = part

    @pl.when(pl.program_id(1) != 0)
    def _():
        o_ref[...] += part


@jax.jit
def kernel(x, W_enc, b_enc, W_dec):
    b, d_in = x.shape
    d_lat = W_enc.shape[0]
    f32 = jnp.float32

    bm1 = min(4096, b)
    bn1 = min(256, d_lat)
    h = pl.pallas_call(
        _enc_kernel,
        grid=(b // bm1, d_lat // bn1),
        in_specs=[
            pl.BlockSpec((bm1, d_in), lambda i, j: (i, 0)),
            pl.BlockSpec((bn1, d_in), lambda i, j: (j, 0)),
            pl.BlockSpec((1, bn1), lambda i, j: (0, j)),
        ],
        out_specs=pl.BlockSpec((bm1, bn1), lambda i, j: (i, j)),
        out_shape=jax.ShapeDtypeStruct((b, d_lat), f32),
        compiler_params=pltpu.CompilerParams(
            dimension_semantics=("parallel", "parallel"),
        ),
    )(x, W_enc, b_enc.reshape(1, d_lat))

    bm2 = min(128, b)
    z = pl.pallas_call(
        functools.partial(_thresh_kernel, k=_TOPK),
        grid=(b // bm2,),
        in_specs=[pl.BlockSpec((bm2, d_lat), lambda i: (i, 0))],
        out_specs=pl.BlockSpec((bm2, d_lat), lambda i: (i, 0)),
        out_shape=jax.ShapeDtypeStruct((b, d_lat), f32),
        compiler_params=pltpu.CompilerParams(
            dimension_semantics=("parallel",),
        ),
    )(h)

    bm4 = min(1024, b)
    bk4 = min(512, d_lat)
    x_hat = pl.pallas_call(
        _dec_kernel,
        grid=(b // bm4, d_lat // bk4),
        in_specs=[
            pl.BlockSpec((bm4, bk4), lambda i, k: (i, k)),
            pl.BlockSpec((d_in, bk4), lambda i, k: (0, k)),
        ],
        out_specs=pl.BlockSpec((bm4, d_in), lambda i, k: (i, 0)),
        out_shape=jax.ShapeDtypeStruct((b, d_in), f32),
        compiler_params=pltpu.CompilerParams(
            dimension_semantics=("parallel", "arbitrary"),
        ),
    )(z, W_dec)

    return x_hat, z
